# baseline (speedup 1.0000x reference)
import jax
import jax.numpy as jnp
from jax.experimental import pallas as pl
from jax.experimental.pallas import tpu as pltpu

_EPS = 1e-8


def _knn_block_kernel(x_ref, w_ref, g_ref, rev_ref, o_ref, feat_ref, rmax_ref,
                      ridx_ref, *, kblk, nblk):
    i = pl.program_id(0)
    q = feat_ref.shape[0]

    @pl.when(i == 0)
    def _init():
        f = jnp.dot(x_ref[...], w_ref[...], preferred_element_type=jnp.float32)
        n = jnp.sqrt(jnp.sum(f * f, axis=1, keepdims=True))
        feat_ref[...] = f / jnp.maximum(n, _EPS)
        rmax_ref[...] = jnp.full(rmax_ref.shape, -jnp.inf, jnp.float32)
        ridx_ref[...] = jnp.zeros(ridx_ref.shape, jnp.int32)

    g = g_ref[...]
    gss = jnp.sum(g * g, axis=1, keepdims=True)
    gn = g * jax.lax.rsqrt(jnp.maximum(gss, _EPS * _EPS))

    nsplit = 8
    w = kblk // nsplit
    feat = feat_ref[...]
    rev = jnp.broadcast_to(rev_ref[0:1, :], (q, w))
    dn = (((1,), (1,)), ((), ()))

    def _scan_part(gn_p):
        sim = jax.lax.dot_general(feat, gn_p, dn,
                                  preferred_element_type=jnp.float32)
        pmax = jnp.max(sim, axis=1, keepdims=True)
        hit = jnp.where(sim >= pmax, rev, 0.0)
        pidx = (w - 1) - jnp.max(hit, axis=1, keepdims=True).astype(jnp.int32)
        return pmax, pidx

    bmax, barg = _scan_part(gn[:w])
    for p in range(1, nsplit):
        pmax, pidx = _scan_part(gn[p * w:(p + 1) * w])
        p_wins = pmax > bmax
        bmax = jnp.where(p_wins, pmax, bmax)
        barg = jnp.where(p_wins, pidx + p * w, barg)
    barg = barg + i * kblk
    better = bmax > rmax_ref[...]
    rmax_ref[...] = jnp.where(better, bmax, rmax_ref[...])
    ridx_ref[...] = jnp.where(better, barg, ridx_ref[...])

    @pl.when(i == nblk - 1)
    def _done():
        o_ref[...] = ridx_ref[...]


def kernel(x, W, training_features):
    q, d_in = x.shape
    d = W.shape[1]
    k_total = training_features.shape[0]
    kblk = 20000
    nblk = pl.cdiv(k_total, kblk)
    if k_total % kblk:
        pad_rows = jnp.broadcast_to(training_features[:1],
                                    (nblk * kblk - k_total, d))
        g = jnp.concatenate([training_features, pad_rows], axis=0)
    else:
        g = training_features

    rev_row = jnp.broadcast_to(
        jnp.arange(kblk // 8 - 1, -1, -1, dtype=jnp.float32)[None, :],
        (8, kblk // 8))

    import functools
    body = functools.partial(_knn_block_kernel, kblk=kblk, nblk=nblk)
    out = pl.pallas_call(
        body,
        grid=(nblk,),
        in_specs=[
            pl.BlockSpec((q, d_in), lambda i: (0, 0)),
            pl.BlockSpec((d_in, d), lambda i: (0, 0)),
            pl.BlockSpec((kblk, d), lambda i: (i, 0)),
            pl.BlockSpec((8, kblk // 8), lambda i: (0, 0)),
        ],
        out_specs=pl.BlockSpec((q, 1), lambda i: (0, 0)),
        out_shape=jax.ShapeDtypeStruct((q, 1), jnp.int32),
        scratch_shapes=[
            pltpu.VMEM((q, d), jnp.float32),
            pltpu.VMEM((q, 1), jnp.float32),
            pltpu.VMEM((q, 1), jnp.int32),
        ],
    )(x, W, g, rev_row)
    return out.reshape(q)


# speedup vs baseline: 1.0356x; 1.0356x over previous
import jax
import jax.numpy as jnp
from jax.experimental import pallas as pl
from jax.experimental.pallas import tpu as pltpu

_EPS = 1e-8


def _knn_block_kernel(x_ref, w_ref, g_ref, rev_ref, o_ref, feat_ref, rmax_ref,
                      ridx_ref, *, kblk, nblk):
    i = pl.program_id(0)
    q = feat_ref.shape[0]

    @pl.when(i == 0)
    def _init():
        f = jnp.dot(x_ref[...], w_ref[...], preferred_element_type=jnp.float32)
        n = jnp.sqrt(jnp.sum(f * f, axis=1, keepdims=True))
        feat_ref[...] = f / jnp.maximum(n, _EPS)
        rmax_ref[...] = jnp.full(rmax_ref.shape, -jnp.inf, jnp.float32)
        ridx_ref[...] = jnp.zeros(ridx_ref.shape, jnp.int32)

    g = g_ref[...]
    gss = jnp.sum(g * g, axis=1, keepdims=True)
    gn = g * jax.lax.rsqrt(jnp.maximum(gss, _EPS * _EPS))

    nsplit = 8
    w = kblk // nsplit
    feat = feat_ref[...]
    rev = jnp.broadcast_to(rev_ref[0:1, :], (q, w))
    dn = (((1,), (1,)), ((), ()))

    def _scan_part(gn_p):
        sim = jax.lax.dot_general(feat, gn_p, dn,
                                  preferred_element_type=jnp.float32)
        pmax = jnp.max(sim, axis=1, keepdims=True)
        hit = jnp.where(sim >= pmax, rev, 0.0)
        pidx = (w - 1) - jnp.max(hit, axis=1, keepdims=True).astype(jnp.int32)
        return pmax, pidx

    bmax, barg = _scan_part(gn[:w])
    for p in range(1, nsplit):
        pmax, pidx = _scan_part(gn[p * w:(p + 1) * w])
        p_wins = pmax > bmax
        bmax = jnp.where(p_wins, pmax, bmax)
        barg = jnp.where(p_wins, pidx + p * w, barg)
    barg = barg + i * kblk
    better = bmax > rmax_ref[...]
    rmax_ref[...] = jnp.where(better, bmax, rmax_ref[...])
    ridx_ref[...] = jnp.where(better, barg, ridx_ref[...])

    @pl.when(i == nblk - 1)
    def _done():
        o_ref[...] = ridx_ref[...]


def kernel(x, W, training_features):
    q, d_in = x.shape
    d = W.shape[1]
    k_total = training_features.shape[0]
    kblk = 10000
    nblk = pl.cdiv(k_total, kblk)
    if k_total % kblk:
        pad_rows = jnp.broadcast_to(training_features[:1],
                                    (nblk * kblk - k_total, d))
        g = jnp.concatenate([training_features, pad_rows], axis=0)
    else:
        g = training_features

    rev_row = jnp.broadcast_to(
        jnp.arange(kblk // 8 - 1, -1, -1, dtype=jnp.float32)[None, :],
        (8, kblk // 8))

    import functools
    body = functools.partial(_knn_block_kernel, kblk=kblk, nblk=nblk)
    out = pl.pallas_call(
        body,
        grid=(nblk,),
        in_specs=[
            pl.BlockSpec((q, d_in), lambda i: (0, 0)),
            pl.BlockSpec((d_in, d), lambda i: (0, 0)),
            pl.BlockSpec((kblk, d), lambda i: (i, 0)),
            pl.BlockSpec((8, kblk // 8), lambda i: (0, 0)),
        ],
        out_specs=pl.BlockSpec((q, 1), lambda i: (0, 0)),
        out_shape=jax.ShapeDtypeStruct((q, 1), jnp.int32),
        scratch_shapes=[
            pltpu.VMEM((q, d), jnp.float32),
            pltpu.VMEM((q, 1), jnp.float32),
            pltpu.VMEM((q, 1), jnp.int32),
        ],
    )(x, W, g, rev_row)
    return out.reshape(q)
